# baseline (speedup 1.0000x reference)
import functools
import math

import jax
import jax.numpy as jnp
from jax import lax
from jax.experimental import pallas as pl
from jax.experimental.pallas import tpu as pltpu
from jax.experimental.pallas import tpu_sc as plsc

_PCT = 0.1
_NC, _NS, _L = 2, 16, 16
_NW = _NC * _NS
_INT_MIN = -(2 ** 31)


def _gat(v, idx):
    return jnp.take_along_axis(v, idx, axis=0)


def _sc_body(n_feat, k, xbits, out, rowbuf, bufa, bufc, hist, hist2,
             in_sem, out_sem):
    int_min = jnp.int32(_INT_MIN)
    lane = jnp.arange(_L, dtype=jnp.int32)
    ones = jnp.ones((_L,), jnp.int32)
    zeros16 = jnp.zeros((_L,), jnp.int32)
    last_idx = jnp.full((_L,), _L - 1, jnp.int32)
    nchunk = n_feat // _L
    rows_per_w = xbits.shape[0] // _NW

    cid = lax.axis_index("c")
    sid = lax.axis_index("s")
    wid = sid * _NC + cid
    row0 = wid * rows_per_w

    def zero_hist():
        for j in range(16):
            hist[pl.ds(j * _L, _L)] = zeros16

    def zero_hist2():
        @plsc.parallel_loop(0, 256, unroll=8)
        def _z(i):
            hist2[pl.ds(i * _L, _L)] = zeros16

    def load_merged(c):
        acc = hist2[pl.ds(c * _L, _L)]
        for l in range(1, 16):
            acc = acc + hist2[pl.ds(l * 256 + c * _L, _L)]
        return acc

    def scan(kprime_v, loader=None):
        def it(c, carry):
            acc_v, found_v, brev_v, j0f_v, cgef_v, accf_v = carry
            t = loader(c) if loader else hist[pl.ds(c * _L, _L)]
            cs = plsc.cumsum(t)
            cge = acc_v + cs
            m = cge >= kprime_v
            pc = plsc.all_reduce_population_count(m)
            j0 = 16 - pc
            fh = jnp.logical_and(found_v == 0, pc > 0)
            brev_v = jnp.where(fh, c * _L + j0, brev_v)
            j0f_v = jnp.where(fh, j0, j0f_v)
            cgef_v = jnp.where(fh, cge, cgef_v)
            accf_v = jnp.where(fh, acc_v, accf_v)
            found_v = jnp.where(pc > 0, jnp.int32(1), found_v)
            acc_v = acc_v + _gat(cs, last_idx)
            return acc_v, found_v, brev_v, j0f_v, cgef_v, accf_v
        init = (zeros16, zeros16, zeros16, zeros16, zeros16, zeros16)
        _, _, brev_v, j0f_v, cgef_v, accf_v = lax.fori_loop(0, 16, it, init,
                                                            unroll=4)
        cnt_gt_v = jnp.where(j0f_v == 0, accf_v,
                             _gat(cgef_v, jnp.maximum(j0f_v - 1, 0)))
        return brev_v, cnt_gt_v

    pltpu.sync_copy(xbits.at[row0], rowbuf)

    def row_fn(r, carry):
        row = row0 + r

        zero_hist2()

        @plsc.parallel_loop(0, nchunk, unroll=8)
        def _sweep_a(i):
            off = i * _L
            v = plsc.bitcast(rowbuf[pl.ds(off, _L)], jnp.int32)
            mag = v & jnp.int32(0x7FFFFFFF)
            u = jnp.where(mag == 0, int_min,
                          jnp.where(v < 0, ~v, v | int_min))
            bufa[pl.ds(off, _L)] = u
            rb = lax.shift_right_logical(~u, 24)
            plsc.addupdate_scatter(hist2, [lane * 256 + rb], ones)

        @pl.when(r < rows_per_w - 1)
        def _():
            pltpu.make_async_copy(xbits.at[row + 1], rowbuf, in_sem).start()

        kprime_v = jnp.full((_L,), k, jnp.int32)
        brev_v, cnt_gt_v = scan(kprime_v, loader=load_merged)
        kprime_v = kprime_v - cnt_gt_v
        prefix_rev_v = brev_v

        for shift in (16, 8, 0):
            zero_hist()

            def _sweep(i, shift=shift, pfx=prefix_rev_v):
                u = bufa[pl.ds(i * _L, _L)]
                nv = ~u
                take = lax.shift_right_logical(nv, shift + 8) == pfx
                rb = lax.shift_right_logical(nv, shift) & jnp.int32(0xFF)
                plsc.addupdate_scatter(hist, [rb], ones, mask=take)
            plsc.parallel_loop(0, nchunk, unroll=16)(_sweep)
            brev_v, cnt_gt_v = scan(kprime_v)
            kprime_v = kprime_v - cnt_gt_v
            prefix_rev_v = lax.shift_left(prefix_rev_v, 8) | brev_v

        key_thr_v = ~prefix_rev_v ^ int_min

        @pl.when(r > 0)
        def _():
            pltpu.make_async_copy(bufc, out.at[row - 1], out_sem).wait()

        @plsc.parallel_loop(0, nchunk, unroll=8)
        def _sweep_m(i):
            off = i * _L
            u = bufa[pl.ds(off, _L)]
            key = u ^ int_min
            bits = jnp.where(key < 0, ~u, key)
            keep = key >= key_thr_v
            bufc[pl.ds(off, _L)] = plsc.bitcast(
                jnp.where(keep, bits, jnp.int32(0)), jnp.float32)

        pltpu.make_async_copy(bufc, out.at[row], out_sem).start()

        @pl.when(r < rows_per_w - 1)
        def _():
            pltpu.make_async_copy(xbits.at[row + 1], rowbuf, in_sem).wait()
        return carry

    lax.fori_loop(0, rows_per_w, row_fn, 0)
    pltpu.make_async_copy(bufc, out.at[row0 + rows_per_w - 1],
                          out_sem).wait()


def kernel(x):
    n_rows, n_feat = x.shape
    k = max(1, math.ceil(n_feat * _PCT))

    mesh = plsc.VectorSubcoreMesh(core_axis_name="c", subcore_axis_name="s",
                                  num_cores=_NC, num_subcores=_NS)
    body = functools.partial(_sc_body, n_feat, k)
    return pl.kernel(
        body,
        out_type=jax.ShapeDtypeStruct((n_rows, n_feat), jnp.float32),
        mesh=mesh,
        scratch_types=[
            pltpu.VMEM((n_feat,), jnp.float32),
            pltpu.VMEM((n_feat,), jnp.int32),
            pltpu.VMEM((n_feat,), jnp.float32),
            pltpu.VMEM((256,), jnp.int32),
            pltpu.VMEM((16 * 256,), jnp.int32),
            pltpu.SemaphoreType.DMA,
            pltpu.SemaphoreType.DMA,
        ],
        compiler_params=pltpu.CompilerParams(needs_layout_passes=False),
    )(x)


# speedup vs baseline: 1.0706x; 1.0706x over previous
import functools
import math

import jax
import jax.numpy as jnp
from jax import lax
from jax.experimental import pallas as pl
from jax.experimental.pallas import tpu as pltpu
from jax.experimental.pallas import tpu_sc as plsc

_PCT = 0.1
_NC, _NS, _L = 2, 16, 16
_NW = _NC * _NS
_INT_MIN = -(2 ** 31)


def _gat(v, idx):
    return jnp.take_along_axis(v, idx, axis=0)


def _sc_body(n_feat, k, xbits, out, rowbuf, bufa, bufc, hist, hist2,
             in_sem, out_sem):
    int_min = jnp.int32(_INT_MIN)
    lane = jnp.arange(_L, dtype=jnp.int32)
    ones = jnp.ones((_L,), jnp.int32)
    zeros16 = jnp.zeros((_L,), jnp.int32)
    last_idx = jnp.full((_L,), _L - 1, jnp.int32)
    nchunk = n_feat // _L
    rows_per_w = xbits.shape[0] // _NW

    cid = lax.axis_index("c")
    sid = lax.axis_index("s")
    wid = sid * _NC + cid
    row0 = wid * rows_per_w

    def zero_hist():
        for j in range(16):
            hist[pl.ds(j * _L, _L)] = zeros16

    def zero_hist2():
        @plsc.parallel_loop(0, 256, unroll=8)
        def _z(i):
            hist2[pl.ds(i * _L, _L)] = zeros16

    def load_merged(c):
        acc = hist2[pl.ds(c * _L, _L)]
        for l in range(1, 16):
            acc = acc + hist2[pl.ds(l * 256 + c * _L, _L)]
        return acc

    def scan(kprime_v, loader=None):
        def it(c, carry):
            acc_v, found_v, brev_v, j0f_v, cgef_v, accf_v = carry
            t = loader(c) if loader else hist[pl.ds(c * _L, _L)]
            cs = plsc.cumsum(t)
            cge = acc_v + cs
            m = cge >= kprime_v
            pc = plsc.all_reduce_population_count(m)
            j0 = 16 - pc
            fh = jnp.logical_and(found_v == 0, pc > 0)
            brev_v = jnp.where(fh, c * _L + j0, brev_v)
            j0f_v = jnp.where(fh, j0, j0f_v)
            cgef_v = jnp.where(fh, cge, cgef_v)
            accf_v = jnp.where(fh, acc_v, accf_v)
            found_v = jnp.where(pc > 0, jnp.int32(1), found_v)
            acc_v = acc_v + _gat(cs, last_idx)
            return acc_v, found_v, brev_v, j0f_v, cgef_v, accf_v
        init = (zeros16, zeros16, zeros16, zeros16, zeros16, zeros16)
        _, _, brev_v, j0f_v, cgef_v, accf_v = lax.fori_loop(0, 16, it, init)
        cnt_gt_v = jnp.where(j0f_v == 0, accf_v,
                             _gat(cgef_v, jnp.maximum(j0f_v - 1, 0)))
        return brev_v, cnt_gt_v

    pltpu.sync_copy(xbits.at[row0], rowbuf)

    def row_fn(r, carry):
        row = row0 + r

        zero_hist2()

        @plsc.parallel_loop(0, nchunk, unroll=8)
        def _sweep_a(i):
            off = i * _L
            v = plsc.bitcast(rowbuf[pl.ds(off, _L)], jnp.int32)
            mag = v & jnp.int32(0x7FFFFFFF)
            u = jnp.where(mag == 0, int_min,
                          jnp.where(v < 0, ~v, v | int_min))
            bufa[pl.ds(off, _L)] = u
            rb = lax.shift_right_logical(~u, 24)
            plsc.addupdate_scatter(hist2, [lane * 256 + rb], ones)

        @pl.when(r < rows_per_w - 1)
        def _():
            pltpu.make_async_copy(xbits.at[row + 1], rowbuf, in_sem).start()

        kprime_v = jnp.full((_L,), k, jnp.int32)
        brev_v, cnt_gt_v = scan(kprime_v, loader=load_merged)
        kprime_v = kprime_v - cnt_gt_v
        prefix_rev_v = brev_v

        for shift in (16, 8, 0):
            zero_hist()

            def _sweep(i, shift=shift, pfx=prefix_rev_v):
                u = bufa[pl.ds(i * _L, _L)]
                nv = ~u
                take = lax.shift_right_logical(nv, shift + 8) == pfx
                rb = lax.shift_right_logical(nv, shift) & jnp.int32(0xFF)
                plsc.addupdate_scatter(hist, [rb], ones, mask=take)
            plsc.parallel_loop(0, nchunk, unroll=8)(_sweep)
            brev_v, cnt_gt_v = scan(kprime_v)
            kprime_v = kprime_v - cnt_gt_v
            prefix_rev_v = lax.shift_left(prefix_rev_v, 8) | brev_v

        key_thr_v = ~prefix_rev_v ^ int_min

        @pl.when(r > 0)
        def _():
            pltpu.make_async_copy(bufc, out.at[row - 1], out_sem).wait()

        @plsc.parallel_loop(0, nchunk, unroll=8)
        def _sweep_m(i):
            off = i * _L
            u = bufa[pl.ds(off, _L)]
            key = u ^ int_min
            bits = jnp.where(key < 0, ~u, key)
            keep = key >= key_thr_v
            bufc[pl.ds(off, _L)] = plsc.bitcast(
                jnp.where(keep, bits, jnp.int32(0)), jnp.float32)

        pltpu.make_async_copy(bufc, out.at[row], out_sem).start()

        @pl.when(r < rows_per_w - 1)
        def _():
            pltpu.make_async_copy(xbits.at[row + 1], rowbuf, in_sem).wait()
        return carry

    lax.fori_loop(0, rows_per_w, row_fn, 0)
    pltpu.make_async_copy(bufc, out.at[row0 + rows_per_w - 1],
                          out_sem).wait()


def kernel(x):
    n_rows, n_feat = x.shape
    k = max(1, math.ceil(n_feat * _PCT))

    mesh = plsc.VectorSubcoreMesh(core_axis_name="c", subcore_axis_name="s",
                                  num_cores=_NC, num_subcores=_NS)
    body = functools.partial(_sc_body, n_feat, k)
    return pl.kernel(
        body,
        out_type=jax.ShapeDtypeStruct((n_rows, n_feat), jnp.float32),
        mesh=mesh,
        scratch_types=[
            pltpu.VMEM((n_feat,), jnp.float32),
            pltpu.VMEM((n_feat,), jnp.int32),
            pltpu.VMEM((n_feat,), jnp.float32),
            pltpu.VMEM((256,), jnp.int32),
            pltpu.VMEM((16 * 256,), jnp.int32),
            pltpu.SemaphoreType.DMA,
            pltpu.SemaphoreType.DMA,
        ],
        compiler_params=pltpu.CompilerParams(needs_layout_passes=False),
    )(x)


# speedup vs baseline: 1.5084x; 1.4089x over previous
import functools
import math

import jax
import jax.numpy as jnp
from jax import lax
from jax.experimental import pallas as pl
from jax.experimental.pallas import tpu as pltpu
from jax.experimental.pallas import tpu_sc as plsc

_PCT = 0.1
_NC, _NS, _L = 2, 16, 16
_NW = _NC * _NS
_INT_MIN = -(2 ** 31)


def _gat(v, idx):
    return jnp.take_along_axis(v, idx, axis=0)


def _sc_body(n_feat, k, xbits, out, rowbuf, bufa, bufc, hist,
             in_sem, out_sem):
    int_min = jnp.int32(_INT_MIN)
    ones = jnp.ones((_L,), jnp.int32)
    zeros16 = jnp.zeros((_L,), jnp.int32)
    last_idx = jnp.full((_L,), _L - 1, jnp.int32)
    nchunk = n_feat // _L
    rows_per_w = xbits.shape[0] // _NW

    cid = lax.axis_index("c")
    sid = lax.axis_index("s")
    wid = sid * _NC + cid
    row0 = wid * rows_per_w

    def zero_hist(nb):
        @plsc.parallel_loop(0, nb // _L, unroll=8)
        def _z(i):
            hist[pl.ds(i * _L, _L)] = zeros16

    def scan(kprime_v, nb):
        def it(c, carry):
            acc_v, found_v, brev_v, j0f_v, cgef_v, accf_v = carry
            t = hist[pl.ds(c * _L, _L)]
            cs = plsc.cumsum(t)
            cge = acc_v + cs
            m = cge >= kprime_v
            pc = plsc.all_reduce_population_count(m)
            j0 = 16 - pc
            fh = jnp.logical_and(found_v == 0, pc > 0)
            brev_v = jnp.where(fh, c * _L + j0, brev_v)
            j0f_v = jnp.where(fh, j0, j0f_v)
            cgef_v = jnp.where(fh, cge, cgef_v)
            accf_v = jnp.where(fh, acc_v, accf_v)
            found_v = jnp.where(pc > 0, jnp.int32(1), found_v)
            acc_v = acc_v + _gat(cs, last_idx)
            return acc_v, found_v, brev_v, j0f_v, cgef_v, accf_v
        init = (zeros16, zeros16, zeros16, zeros16, zeros16, zeros16)
        _, _, brev_v, j0f_v, cgef_v, accf_v = lax.fori_loop(0, nb // _L, it,
                                                            init)
        cnt_gt_v = jnp.where(j0f_v == 0, accf_v,
                             _gat(cgef_v, jnp.maximum(j0f_v - 1, 0)))
        return brev_v, cnt_gt_v

    pltpu.sync_copy(xbits.at[row0], rowbuf)

    def row_fn(r, carry):
        row = row0 + r

        zero_hist(2048)

        @plsc.parallel_loop(0, nchunk, unroll=8)
        def _sweep_a(i):
            off = i * _L
            v = plsc.bitcast(rowbuf[pl.ds(off, _L)], jnp.int32)
            mag = v & jnp.int32(0x7FFFFFFF)
            u = jnp.where(mag == 0, int_min,
                          jnp.where(v < 0, ~v, v | int_min))
            bufa[pl.ds(off, _L)] = u
            rb = lax.shift_right_logical(~u, 21)
            plsc.addupdate_scatter(hist, [rb], ones)

        @pl.when(r < rows_per_w - 1)
        def _():
            pltpu.make_async_copy(xbits.at[row + 1], rowbuf, in_sem).start()

        kprime_v = jnp.full((_L,), k, jnp.int32)
        brev_v, cnt_gt_v = scan(kprime_v, 2048)
        kprime_v = kprime_v - cnt_gt_v
        prefix_rev_v = brev_v

        for shift, width, nb in ((10, 11, 2048), (0, 10, 1024)):
            zero_hist(nb)
            fmask = jnp.int32(nb - 1)

            def _sweep(i, shift=shift, width=width, fmask=fmask,
                       pfx=prefix_rev_v):
                u = bufa[pl.ds(i * _L, _L)]
                nv = ~u
                take = lax.shift_right_logical(nv, shift + width) == pfx
                rb = lax.shift_right_logical(nv, shift) & fmask
                plsc.addupdate_scatter(hist, [rb], ones, mask=take)
            plsc.parallel_loop(0, nchunk, unroll=8)(_sweep)
            brev_v, cnt_gt_v = scan(kprime_v, nb)
            kprime_v = kprime_v - cnt_gt_v
            prefix_rev_v = lax.shift_left(prefix_rev_v, width) | brev_v

        key_thr_v = ~prefix_rev_v ^ int_min

        @pl.when(r > 0)
        def _():
            pltpu.make_async_copy(bufc, out.at[row - 1], out_sem).wait()

        @plsc.parallel_loop(0, nchunk, unroll=8)
        def _sweep_m(i):
            off = i * _L
            u = bufa[pl.ds(off, _L)]
            key = u ^ int_min
            bits = jnp.where(key < 0, ~u, key)
            keep = key >= key_thr_v
            bufc[pl.ds(off, _L)] = plsc.bitcast(
                jnp.where(keep, bits, jnp.int32(0)), jnp.float32)

        pltpu.make_async_copy(bufc, out.at[row], out_sem).start()

        @pl.when(r < rows_per_w - 1)
        def _():
            pltpu.make_async_copy(xbits.at[row + 1], rowbuf, in_sem).wait()
        return carry

    lax.fori_loop(0, rows_per_w, row_fn, 0)
    pltpu.make_async_copy(bufc, out.at[row0 + rows_per_w - 1],
                          out_sem).wait()


def kernel(x):
    n_rows, n_feat = x.shape
    k = max(1, math.ceil(n_feat * _PCT))

    mesh = plsc.VectorSubcoreMesh(core_axis_name="c", subcore_axis_name="s",
                                  num_cores=_NC, num_subcores=_NS)
    body = functools.partial(_sc_body, n_feat, k)
    return pl.kernel(
        body,
        out_type=jax.ShapeDtypeStruct((n_rows, n_feat), jnp.float32),
        mesh=mesh,
        scratch_types=[
            pltpu.VMEM((n_feat,), jnp.float32),
            pltpu.VMEM((n_feat,), jnp.int32),
            pltpu.VMEM((n_feat,), jnp.float32),
            pltpu.VMEM((2048,), jnp.int32),
            pltpu.SemaphoreType.DMA,
            pltpu.SemaphoreType.DMA,
        ],
        compiler_params=pltpu.CompilerParams(needs_layout_passes=False),
    )(x)


# speedup vs baseline: 1.5212x; 1.0085x over previous
import functools
import math

import jax
import jax.numpy as jnp
from jax import lax
from jax.experimental import pallas as pl
from jax.experimental.pallas import tpu as pltpu
from jax.experimental.pallas import tpu_sc as plsc

_PCT = 0.1
_NC, _NS, _L = 2, 16, 16
_NW = _NC * _NS
_INT_MIN = -(2 ** 31)


def _gat(v, idx):
    return jnp.take_along_axis(v, idx, axis=0)


def _sc_body(n_feat, k, xbits, out, rowbuf, bufa, bufc, hist,
             in_sem, out_sem):
    int_min = jnp.int32(_INT_MIN)
    ones = jnp.ones((_L,), jnp.int32)
    zeros16 = jnp.zeros((_L,), jnp.int32)
    last_idx = jnp.full((_L,), _L - 1, jnp.int32)
    nchunk = n_feat // _L
    rows_per_w = xbits.shape[0] // _NW

    cid = lax.axis_index("c")
    sid = lax.axis_index("s")
    wid = sid * _NC + cid
    row0 = wid * rows_per_w

    def zero_hist(nb):
        @plsc.parallel_loop(0, nb // _L, unroll=8)
        def _z(i):
            hist[pl.ds(i * _L, _L)] = zeros16

    def scan(kprime_v, nb):
        U = 4
        def it(cq, carry):
            acc_v, found_v, brev_v, j0f_v, cgef_v, accf_v = carry
            base = cq * U
            css = [plsc.cumsum(hist[pl.ds((base + uu) * _L, _L)])
                   for uu in range(U)]
            accs = [acc_v]
            for uu in range(U):
                accs.append(accs[-1] + _gat(css[uu], last_idx))
            for uu in range(U):
                cge = accs[uu] + css[uu]
                m = cge >= kprime_v
                pc = plsc.all_reduce_population_count(m)
                j0 = 16 - pc
                fh = jnp.logical_and(found_v == 0, pc > 0)
                brev_v = jnp.where(fh, (base + uu) * _L + j0, brev_v)
                j0f_v = jnp.where(fh, j0, j0f_v)
                cgef_v = jnp.where(fh, cge, cgef_v)
                accf_v = jnp.where(fh, accs[uu], accf_v)
                found_v = jnp.where(pc > 0, jnp.int32(1), found_v)
            return accs[U], found_v, brev_v, j0f_v, cgef_v, accf_v
        init = (zeros16, zeros16, zeros16, zeros16, zeros16, zeros16)
        _, _, brev_v, j0f_v, cgef_v, accf_v = lax.fori_loop(
            0, nb // _L // U, it, init)
        cnt_gt_v = jnp.where(j0f_v == 0, accf_v,
                             _gat(cgef_v, jnp.maximum(j0f_v - 1, 0)))
        return brev_v, cnt_gt_v

    pltpu.sync_copy(xbits.at[row0], rowbuf)

    def row_fn(r, carry):
        row = row0 + r

        zero_hist(2048)

        @plsc.parallel_loop(0, nchunk, unroll=8)
        def _sweep_a(i):
            off = i * _L
            v = plsc.bitcast(rowbuf[pl.ds(off, _L)], jnp.int32)
            mag = v & jnp.int32(0x7FFFFFFF)
            u = jnp.where(mag == 0, int_min,
                          jnp.where(v < 0, ~v, v | int_min))
            bufa[pl.ds(off, _L)] = u
            rb = lax.shift_right_logical(~u, 21)
            plsc.addupdate_scatter(hist, [rb], ones)

        @pl.when(r < rows_per_w - 1)
        def _():
            pltpu.make_async_copy(xbits.at[row + 1], rowbuf, in_sem).start()

        kprime_v = jnp.full((_L,), k, jnp.int32)
        brev_v, cnt_gt_v = scan(kprime_v, 2048)
        kprime_v = kprime_v - cnt_gt_v
        prefix_rev_v = brev_v

        for shift, width, nb in ((10, 11, 2048), (0, 10, 1024)):
            zero_hist(nb)
            fmask = jnp.int32(nb - 1)

            def _sweep(i, shift=shift, width=width, fmask=fmask,
                       pfx=prefix_rev_v):
                u = bufa[pl.ds(i * _L, _L)]
                nv = ~u
                take = lax.shift_right_logical(nv, shift + width) == pfx
                rb = lax.shift_right_logical(nv, shift) & fmask
                plsc.addupdate_scatter(hist, [rb], ones, mask=take)
            plsc.parallel_loop(0, nchunk, unroll=8)(_sweep)
            brev_v, cnt_gt_v = scan(kprime_v, nb)
            kprime_v = kprime_v - cnt_gt_v
            prefix_rev_v = lax.shift_left(prefix_rev_v, width) | brev_v

        key_thr_v = ~prefix_rev_v ^ int_min

        @pl.when(r > 0)
        def _():
            pltpu.make_async_copy(bufc, out.at[row - 1], out_sem).wait()

        @plsc.parallel_loop(0, nchunk, unroll=8)
        def _sweep_m(i):
            off = i * _L
            u = bufa[pl.ds(off, _L)]
            key = u ^ int_min
            bits = jnp.where(key < 0, ~u, key)
            keep = key >= key_thr_v
            bufc[pl.ds(off, _L)] = plsc.bitcast(
                jnp.where(keep, bits, jnp.int32(0)), jnp.float32)

        pltpu.make_async_copy(bufc, out.at[row], out_sem).start()

        @pl.when(r < rows_per_w - 1)
        def _():
            pltpu.make_async_copy(xbits.at[row + 1], rowbuf, in_sem).wait()
        return carry

    lax.fori_loop(0, rows_per_w, row_fn, 0)
    pltpu.make_async_copy(bufc, out.at[row0 + rows_per_w - 1],
                          out_sem).wait()


def kernel(x):
    n_rows, n_feat = x.shape
    k = max(1, math.ceil(n_feat * _PCT))

    mesh = plsc.VectorSubcoreMesh(core_axis_name="c", subcore_axis_name="s",
                                  num_cores=_NC, num_subcores=_NS)
    body = functools.partial(_sc_body, n_feat, k)
    return pl.kernel(
        body,
        out_type=jax.ShapeDtypeStruct((n_rows, n_feat), jnp.float32),
        mesh=mesh,
        scratch_types=[
            pltpu.VMEM((n_feat,), jnp.float32),
            pltpu.VMEM((n_feat,), jnp.int32),
            pltpu.VMEM((n_feat,), jnp.float32),
            pltpu.VMEM((2048,), jnp.int32),
            pltpu.SemaphoreType.DMA,
            pltpu.SemaphoreType.DMA,
        ],
        compiler_params=pltpu.CompilerParams(needs_layout_passes=False),
    )(x)
